# Initial kernel scaffold; baseline (speedup 1.0000x reference)
import jax
import jax.numpy as jnp
from jax.experimental import pallas as pl


def kernel(x, weight):
    raise NotImplementedError("write your pallas kernel here")


# speedup vs baseline: 177.5822x; 177.5822x over previous
import functools

import jax
import jax.numpy as jnp
from jax import lax
from jax.experimental import pallas as pl
from jax.experimental.pallas import tpu as pltpu
from jax.experimental.pallas import tpu_sc as plsc

_N = 4096
_S = 200
_V = 100000
_NC = 2
_NS = 16
_L = 16
_NW = _NC * _NS
_ROWS = _N // _NW
_BLOCKS = _ROWS // _L

_mesh = plsc.VectorSubcoreMesh(core_axis_name="c", subcore_axis_name="s")


def _bow_logits_body(x_hbm, w_hbm, out_hbm, table_v, idx_v, out_v):
    wid = lax.axis_index("s") * _NC + lax.axis_index("c")
    base_row = wid * _ROWS
    pltpu.sync_copy(w_hbm, table_v)
    pltpu.sync_copy(x_hbm.at[pl.ds(base_row * _S, _ROWS * _S)], idx_v)
    lane = lax.iota(jnp.int32, _L)
    for b in range(_BLOCKS):
        row_off = (b * _L + lane) * _S

        def body(s, acc, row_off=row_off):
            idx16 = plsc.load_gather(idx_v, [row_off + s])
            vals = plsc.load_gather(table_v, [idx16])
            return acc + vals

        acc = lax.fori_loop(0, _S, body, jnp.zeros((_L,), jnp.float32))
        out_v[pl.ds(b * _L, _L)] = acc
    pltpu.sync_copy(out_v, out_hbm.at[pl.ds(base_row, _ROWS)])


_bow_logits = functools.partial(
    pl.kernel,
    mesh=_mesh,
    out_type=jax.ShapeDtypeStruct((_N,), jnp.float32),
    scratch_types=[
        pltpu.VMEM((_V,), jnp.float32),
        pltpu.VMEM((_ROWS * _S,), jnp.int32),
        pltpu.VMEM((_ROWS,), jnp.float32),
    ],
    compiler_params=pltpu.CompilerParams(needs_layout_passes=False),
)(_bow_logits_body)


def kernel(x, weight):
    xf = x.reshape(-1)
    wf = weight.reshape(-1)
    return _bow_logits(xf, wf)


# speedup vs baseline: 208.5464x; 1.1744x over previous
import functools

import jax
import jax.numpy as jnp
from jax import lax
from jax.experimental import pallas as pl
from jax.experimental.pallas import tpu as pltpu
from jax.experimental.pallas import tpu_sc as plsc

_N = 4096
_S = 200
_V = 100000
_NC = 2
_NS = 16
_L = 16
_NW = _NC * _NS
_ROWS = _N // _NW
_BLOCKS = _ROWS // _L

_mesh = plsc.VectorSubcoreMesh(core_axis_name="c", subcore_axis_name="s")


def _bow_logits_body(x_hbm, w_hbm, out_hbm, table_v, idx_v, out_v,
                     sem_w, sem_x):
    wid = lax.axis_index("s") * _NC + lax.axis_index("c")
    base_row = wid * _ROWS
    cp_w = pltpu.make_async_copy(w_hbm, table_v, sem_w)
    cp_x = pltpu.make_async_copy(
        x_hbm.at[pl.ds(base_row * _S, _ROWS * _S)], idx_v, sem_x)
    cp_w.start()
    cp_x.start()
    cp_w.wait()
    cp_x.wait()

    lane = lax.iota(jnp.int32, _L)
    row_offs = [(b * _L + lane) * _S for b in range(_BLOCKS)]
    zero = jnp.zeros((_L,), jnp.float32)

    @plsc.parallel_loop(0, _S, unroll=2, carry=(zero,) * _BLOCKS)
    def accs(s, acc):
        new = []
        for b in range(_BLOCKS):
            idx16 = plsc.load_gather(idx_v, [row_offs[b] + s])
            new.append(acc[b] + plsc.load_gather(table_v, [idx16]))
        return tuple(new)

    for b in range(_BLOCKS):
        out_v[pl.ds(b * _L, _L)] = accs[b]
    pltpu.sync_copy(out_v, out_hbm.at[pl.ds(base_row, _ROWS)])


_bow_logits = functools.partial(
    pl.kernel,
    mesh=_mesh,
    out_type=jax.ShapeDtypeStruct((_N,), jnp.float32),
    scratch_types=[
        pltpu.VMEM((_V,), jnp.float32),
        pltpu.VMEM((_ROWS * _S,), jnp.int32),
        pltpu.VMEM((_ROWS,), jnp.float32),
        pltpu.SemaphoreType.DMA,
        pltpu.SemaphoreType.DMA,
    ],
    compiler_params=pltpu.CompilerParams(needs_layout_passes=False),
)(_bow_logits_body)


def kernel(x, weight):
    xf = x.reshape(-1)
    wf = weight.reshape(-1)
    return _bow_logits(xf, wf)


# speedup vs baseline: 242.4179x; 1.1624x over previous
import functools

import jax
import jax.numpy as jnp
from jax import lax
from jax.experimental import pallas as pl
from jax.experimental.pallas import tpu as pltpu
from jax.experimental.pallas import tpu_sc as plsc

_N = 4096
_S = 200
_V = 100000
_NC = 2
_NS = 16
_L = 16
_NW = _NC * _NS
_ROWS = _N // _NW
_BLOCKS = _ROWS // _L
_VH = _V // 2

_mesh = plsc.VectorSubcoreMesh(core_axis_name="c", subcore_axis_name="s")


def _bow_logits_body(x_hbm, w_hbm, out_hbm, table_sh, table_v, idx_v, out_v,
                     sem_x):
    sid = lax.axis_index("s")
    wid = sid * _NC + lax.axis_index("c")
    base_row = wid * _ROWS
    cp_x = pltpu.make_async_copy(
        x_hbm.at[pl.ds(base_row * _S, _ROWS * _S)], idx_v, sem_x)
    cp_x.start()

    for h in range(2):
        @pl.when(sid == 0)
        def _(h=h):
            pltpu.sync_copy(w_hbm.at[h], table_sh)

        plsc.subcore_barrier()
        pltpu.sync_copy(table_sh, table_v.at[pl.ds(h * _VH, _VH)])
        plsc.subcore_barrier()
    cp_x.wait()

    lane = lax.iota(jnp.int32, _L)
    row_offs = [(b * _L + lane) * _S for b in range(_BLOCKS)]
    zero = jnp.zeros((_L,), jnp.float32)

    @plsc.parallel_loop(0, _S, unroll=2, carry=(zero,) * _BLOCKS)
    def accs(s, acc):
        new = []
        for b in range(_BLOCKS):
            idx16 = plsc.load_gather(idx_v, [row_offs[b] + s])
            new.append(acc[b] + plsc.load_gather(table_v, [idx16]))
        return tuple(new)

    for b in range(_BLOCKS):
        out_v[pl.ds(b * _L, _L)] = accs[b]
    pltpu.sync_copy(out_v, out_hbm.at[pl.ds(base_row, _ROWS)])


_bow_logits = functools.partial(
    pl.kernel,
    mesh=_mesh,
    out_type=jax.ShapeDtypeStruct((_N,), jnp.float32),
    scratch_types=[
        pltpu.VMEM_SHARED((_VH,), jnp.float32),
        pltpu.VMEM((_V,), jnp.float32),
        pltpu.VMEM((_ROWS * _S,), jnp.int32),
        pltpu.VMEM((_ROWS,), jnp.float32),
        pltpu.SemaphoreType.DMA,
    ],
    compiler_params=pltpu.CompilerParams(needs_layout_passes=False),
)(_bow_logits_body)


def kernel(x, weight):
    xf = x.reshape(-1)
    wf = weight.reshape(2, _VH)
    return _bow_logits(xf, wf)
